# Initial kernel scaffold; baseline (speedup 1.0000x reference)
import functools

import jax
import jax.numpy as jnp
from jax import lax
from jax.experimental import pallas as pl
from jax.experimental.pallas import tpu as pltpu
from jax.experimental.pallas import tpu_sc as plsc

_EMBED = 128
_BATCH = 16384
_MARGIN = 1.0
_LANES = 16
_NSEG = _EMBED // _LANES

_NC = 2
_NS = 16
_NW = _NC * _NS
_B_PER_W = _BATCH // _NW
_CHUNK = 128
_NCHUNK = _B_PER_W // _CHUNK


def _tec_kernel(pos_hbm, neg_hbm, ent_hbm, rel_hbm, out_hbm,
                idx_ph, idx_pr, idx_pt, idx_nh, idx_nr, idx_nt,
                ph, pr, pt, nh, nr, nt, out_v, sem):
    wid = lax.axis_index("s") * _NC + lax.axis_index("c")
    base0 = wid * _B_PER_W

    def chunk_body(ci, loss_acc):
        base = base0 + ci * _CHUNK
        sl = pl.ds(base, _CHUNK)
        pltpu.sync_copy(pos_hbm.at[0, sl], idx_ph)
        pltpu.sync_copy(pos_hbm.at[1, sl], idx_pr)
        pltpu.sync_copy(pos_hbm.at[2, sl], idx_pt)
        pltpu.sync_copy(neg_hbm.at[0, sl], idx_nh)
        pltpu.sync_copy(neg_hbm.at[1, sl], idx_nr)
        pltpu.sync_copy(neg_hbm.at[2, sl], idx_nt)

        c1 = pltpu.async_copy(ent_hbm.at[idx_ph], ph, sem)
        c2 = pltpu.async_copy(rel_hbm.at[idx_pr], pr, sem)
        c3 = pltpu.async_copy(ent_hbm.at[idx_pt], pt, sem)
        c4 = pltpu.async_copy(ent_hbm.at[idx_nh], nh, sem)
        c5 = pltpu.async_copy(rel_hbm.at[idx_nr], nr, sem)
        c6 = pltpu.async_copy(ent_hbm.at[idx_nt], nt, sem)
        c1.wait()
        c2.wait()
        c3.wait()
        c4.wait()
        c5.wait()
        c6.wait()

        def row_body(b, acc):
            d = jnp.zeros((_LANES,), jnp.float32)
            for j in range(_NSEG):
                ds = pl.ds(j * _LANES, _LANES)
                pd = jnp.abs(ph[b, ds] + pr[b, ds] - pt[b, ds])
                nd = jnp.abs(nh[b, ds] + nr[b, ds] - nt[b, ds])
                d = d + (nd - pd)
            s = jnp.sum(d) + _MARGIN
            return acc + jnp.maximum(s, 0.0)

        return lax.fori_loop(0, _CHUNK, row_body, loss_acc)

    loss = lax.fori_loop(0, _NCHUNK, chunk_body, jnp.float32(0.0))

    lane = lax.broadcasted_iota(jnp.int32, (_LANES,), 0)
    out_v[...] = jnp.where(lane == 0, loss, 0.0)
    pltpu.sync_copy(out_v, out_hbm.at[wid])


@jax.jit
def kernel(pos_exmpl, neg_exmpl, entity_emb, relation_emb):
    mesh = plsc.VectorSubcoreMesh(core_axis_name="c", subcore_axis_name="s")
    run = functools.partial(
        pl.kernel,
        mesh=mesh,
        out_type=jax.ShapeDtypeStruct((_NW, _LANES), jnp.float32),
        scratch_types=[
            pltpu.VMEM((_CHUNK,), jnp.int32),
            pltpu.VMEM((_CHUNK,), jnp.int32),
            pltpu.VMEM((_CHUNK,), jnp.int32),
            pltpu.VMEM((_CHUNK,), jnp.int32),
            pltpu.VMEM((_CHUNK,), jnp.int32),
            pltpu.VMEM((_CHUNK,), jnp.int32),
            pltpu.VMEM((_CHUNK, _EMBED), jnp.float32),
            pltpu.VMEM((_CHUNK, _EMBED), jnp.float32),
            pltpu.VMEM((_CHUNK, _EMBED), jnp.float32),
            pltpu.VMEM((_CHUNK, _EMBED), jnp.float32),
            pltpu.VMEM((_CHUNK, _EMBED), jnp.float32),
            pltpu.VMEM((_CHUNK, _EMBED), jnp.float32),
            pltpu.VMEM((_LANES,), jnp.float32),
            pltpu.SemaphoreType.DMA,
        ],
    )(_tec_kernel)
    partials = run(pos_exmpl, neg_exmpl, entity_emb, relation_emb)
    return jnp.sum(partials)


# speedup vs baseline: 1.5847x; 1.5847x over previous
import functools

import jax
import jax.numpy as jnp
from jax import lax
from jax.experimental import pallas as pl
from jax.experimental.pallas import tpu as pltpu
from jax.experimental.pallas import tpu_sc as plsc

_EMBED = 128
_BATCH = 16384
_MARGIN = 1.0
_LANES = 16
_NSEG = _EMBED // _LANES

_NC = 2
_NS = 16
_NW = _NC * _NS
_B_PER_W = _BATCH // _NW
_CHUNK = 128
_NCHUNK = _B_PER_W // _CHUNK


def _tec_kernel(pos_hbm, neg_hbm, ent_hbm, rel_hbm, out_hbm,
                idx_ph, idx_pr, idx_pt, idx_nh, idx_nr, idx_nt,
                ph, pr, pt, nh, nr, nt, out_v, sem):
    wid = lax.axis_index("s") * _NC + lax.axis_index("c")
    base0 = wid * _B_PER_W

    def chunk_body(ci, loss_acc):
        base = base0 + ci * _CHUNK
        pltpu.sync_copy(pos_hbm.at[pl.ds(base, _CHUNK)], idx_ph)
        pltpu.sync_copy(pos_hbm.at[pl.ds(_BATCH + base, _CHUNK)], idx_pr)
        pltpu.sync_copy(pos_hbm.at[pl.ds(2 * _BATCH + base, _CHUNK)], idx_pt)
        pltpu.sync_copy(neg_hbm.at[pl.ds(base, _CHUNK)], idx_nh)
        pltpu.sync_copy(neg_hbm.at[pl.ds(_BATCH + base, _CHUNK)], idx_nr)
        pltpu.sync_copy(neg_hbm.at[pl.ds(2 * _BATCH + base, _CHUNK)], idx_nt)

        c1 = pltpu.async_copy(ent_hbm.at[idx_ph], ph, sem)
        c2 = pltpu.async_copy(rel_hbm.at[idx_pr], pr, sem)
        c3 = pltpu.async_copy(ent_hbm.at[idx_pt], pt, sem)
        c4 = pltpu.async_copy(ent_hbm.at[idx_nh], nh, sem)
        c5 = pltpu.async_copy(rel_hbm.at[idx_nr], nr, sem)
        c6 = pltpu.async_copy(ent_hbm.at[idx_nt], nt, sem)
        c1.wait()
        c2.wait()
        c3.wait()
        c4.wait()
        c5.wait()
        c6.wait()

        lane = lax.broadcasted_iota(jnp.int32, (_LANES,), 0)

        def row_body(b, acc):
            d = jnp.zeros((_LANES,), jnp.float32)
            for j in range(_NSEG):
                ds = pl.ds(j * _LANES, _LANES)
                pd = jnp.abs(ph[b, ds] + pr[b, ds] - pt[b, ds])
                nd = jnp.abs(nh[b, ds] + nr[b, ds] - nt[b, ds])
                d = d + (nd - pd)
            s = d
            for k in (1, 2, 4, 8):
                s = s + s.at[lane ^ k].get(mode="promise_in_bounds")
            c = jnp.maximum(s + _MARGIN, 0.0)
            return acc + jnp.where(lane == 0, c, 0.0)

        return lax.fori_loop(0, _CHUNK, row_body, loss_acc)

    loss = lax.fori_loop(0, _NCHUNK, chunk_body,
                         jnp.zeros((_LANES,), jnp.float32))
    out_v[...] = loss
    pltpu.sync_copy(out_v, out_hbm.at[wid])


@jax.jit
def kernel(pos_exmpl, neg_exmpl, entity_emb, relation_emb):
    mesh = plsc.VectorSubcoreMesh(core_axis_name="c", subcore_axis_name="s")
    run = functools.partial(
        pl.kernel,
        mesh=mesh,
        out_type=jax.ShapeDtypeStruct((_NW, _LANES), jnp.float32),
        scratch_types=[
            pltpu.VMEM((_CHUNK,), jnp.int32),
            pltpu.VMEM((_CHUNK,), jnp.int32),
            pltpu.VMEM((_CHUNK,), jnp.int32),
            pltpu.VMEM((_CHUNK,), jnp.int32),
            pltpu.VMEM((_CHUNK,), jnp.int32),
            pltpu.VMEM((_CHUNK,), jnp.int32),
            pltpu.VMEM((_CHUNK, _EMBED), jnp.float32),
            pltpu.VMEM((_CHUNK, _EMBED), jnp.float32),
            pltpu.VMEM((_CHUNK, _EMBED), jnp.float32),
            pltpu.VMEM((_CHUNK, _EMBED), jnp.float32),
            pltpu.VMEM((_CHUNK, _EMBED), jnp.float32),
            pltpu.VMEM((_CHUNK, _EMBED), jnp.float32),
            pltpu.VMEM((_LANES,), jnp.float32),
            pltpu.SemaphoreType.DMA,
        ],
    )(_tec_kernel)
    partials = run(pos_exmpl.reshape(-1), neg_exmpl.reshape(-1),
                   entity_emb, relation_emb)
    return jnp.sum(partials)


# speedup vs baseline: 2.2558x; 1.4234x over previous
import functools

import jax
import jax.numpy as jnp
from jax import lax
from jax.experimental import pallas as pl
from jax.experimental.pallas import tpu as pltpu
from jax.experimental.pallas import tpu_sc as plsc

_EMBED = 128
_BATCH = 16384
_MARGIN = 1.0
_LANES = 16
_NSEG = _EMBED // _LANES

_NC = 2
_NS = 16
_NW = _NC * _NS
_B_PER_W = _BATCH // _NW
_CHUNK = 64
_NCHUNK = _B_PER_W // _CHUNK


def _tec_kernel(pos_hbm, neg_hbm, ent_hbm, rel_hbm, out_hbm,
                idx_ph, idx_pr, idx_pt, idx_nh, idx_nr, idx_nt,
                ph0, pr0, pt0, nh0, nr0, nt0,
                ph1, pr1, pt1, nh1, nr1, nt1,
                out_v, sem):
    wid = lax.axis_index("s") * _NC + lax.axis_index("c")
    base0 = wid * _B_PER_W

    pltpu.sync_copy(pos_hbm.at[pl.ds(base0, _B_PER_W)], idx_ph)
    pltpu.sync_copy(pos_hbm.at[pl.ds(_BATCH + base0, _B_PER_W)], idx_pr)
    pltpu.sync_copy(pos_hbm.at[pl.ds(2 * _BATCH + base0, _B_PER_W)], idx_pt)
    pltpu.sync_copy(neg_hbm.at[pl.ds(base0, _B_PER_W)], idx_nh)
    pltpu.sync_copy(neg_hbm.at[pl.ds(_BATCH + base0, _B_PER_W)], idx_nr)
    pltpu.sync_copy(neg_hbm.at[pl.ds(2 * _BATCH + base0, _B_PER_W)], idx_nt)

    tables = (ent_hbm, rel_hbm, ent_hbm, ent_hbm, rel_hbm, ent_hbm)
    idxs = (idx_ph, idx_pr, idx_pt, idx_nh, idx_nr, idx_nt)
    bufsets = ((ph0, pr0, pt0, nh0, nr0, nt0),
               (ph1, pr1, pt1, nh1, nr1, nt1))

    def fire(ci, bufset):
        sl = pl.ds(ci * _CHUNK, _CHUNK)
        return [pltpu.async_copy(tab.at[idx.at[sl]], buf, sem)
                for tab, idx, buf in zip(tables, idxs, bufset)]

    lane = lax.broadcasted_iota(jnp.int32, (_LANES,), 0)

    def compute_chunk(bufset, acc0):
        ph, pr, pt, nh, nr, nt = bufset

        def row_body(b, acc):
            d = jnp.zeros((_LANES,), jnp.float32)
            for j in range(_NSEG):
                ds = pl.ds(j * _LANES, _LANES)
                pd = jnp.abs(ph[b, ds] + pr[b, ds] - pt[b, ds])
                nd = jnp.abs(nh[b, ds] + nr[b, ds] - nt[b, ds])
                d = d + (nd - pd)
            s = d
            for k in (1, 2, 4, 8):
                s = s + s.at[lane ^ k].get(mode="promise_in_bounds")
            c = jnp.maximum(s + _MARGIN, 0.0)
            return acc + jnp.where(lane == 0, c, 0.0)

        return lax.fori_loop(0, _CHUNK, row_body, acc0)

    acc = jnp.zeros((_LANES,), jnp.float32)
    pending = fire(0, bufsets[0])
    for ci in range(_NCHUNK):
        nxt = fire(ci + 1, bufsets[(ci + 1) % 2]) if ci + 1 < _NCHUNK else None
        for cp in pending:
            cp.wait()
        acc = compute_chunk(bufsets[ci % 2], acc)
        pending = nxt

    out_v[...] = acc
    pltpu.sync_copy(out_v, out_hbm.at[wid])


@jax.jit
def kernel(pos_exmpl, neg_exmpl, entity_emb, relation_emb):
    mesh = plsc.VectorSubcoreMesh(core_axis_name="c", subcore_axis_name="s")
    buf = pltpu.VMEM((_CHUNK, _EMBED), jnp.float32)
    run = functools.partial(
        pl.kernel,
        mesh=mesh,
        out_type=jax.ShapeDtypeStruct((_NW, _LANES), jnp.float32),
        scratch_types=(
            [pltpu.VMEM((_B_PER_W,), jnp.int32)] * 6
            + [buf] * 12
            + [pltpu.VMEM((_LANES,), jnp.float32), pltpu.SemaphoreType.DMA]
        ),
    )(_tec_kernel)
    partials = run(pos_exmpl.reshape(-1), neg_exmpl.reshape(-1),
                   entity_emb, relation_emb)
    return jnp.sum(partials)
